# baseline (speedup 1.0000x reference)
import functools

import jax
import jax.numpy as jnp
from jax import lax
from jax.experimental import pallas as pl
from jax.experimental.pallas import tpu as pltpu
from jax.experimental.pallas import tpu_sc as plsc

BATCH = 4096
SEQ = 200
EMBED = 64
HALF = SEQ // 2
NC, NS = 2, 16
NW = NC * NS
BPW = BATCH // NW


def _pool_body(x_hbm, table_hbm, z_hbm, idx_v, rows_a, rows_b, zacc, sem):
    wid = lax.axis_index("s") * NC + lax.axis_index("c")
    base = wid * BPW
    pltpu.sync_copy(x_hbm.at[pl.ds(2 * base, 2 * BPW)], idx_v)

    def per_row(r, carry):
        ca = pltpu.async_copy(table_hbm.at[idx_v.at[2 * r]], rows_a, sem)
        cb = pltpu.async_copy(table_hbm.at[idx_v.at[2 * r + 1]], rows_b, sem)
        ca.wait()
        cb.wait()

        def red(i, acc):
            return tuple(
                acc[c]
                + rows_a[i, pl.ds(16 * c, 16)]
                + rows_b[i, pl.ds(16 * c, 16)]
                for c in range(4)
            )

        zero = jnp.zeros((16,), jnp.float32)
        acc = lax.fori_loop(0, HALF, red, (zero, zero, zero, zero))
        scale = jnp.float32(1.0 / SEQ)
        for c in range(4):
            zacc[r, pl.ds(16 * c, 16)] = acc[c] * scale
        return carry

    lax.fori_loop(0, BPW, per_row, 0)
    pltpu.sync_copy(zacc, z_hbm.at[pl.ds(base, BPW)])


@functools.partial(jax.jit, static_argnames=())
def _pool(x2, table):
    mesh = plsc.VectorSubcoreMesh(core_axis_name="c", subcore_axis_name="s")
    kern = pl.kernel(
        _pool_body,
        out_type=jax.ShapeDtypeStruct((BATCH, EMBED), jnp.float32),
        mesh=mesh,
        scratch_types=[
            pltpu.VMEM((2 * BPW, HALF), jnp.int32),
            pltpu.VMEM((HALF, EMBED), jnp.float32),
            pltpu.VMEM((HALF, EMBED), jnp.float32),
            pltpu.VMEM((BPW, EMBED), jnp.float32),
            pltpu.SemaphoreType.DMA,
        ],
        compiler_params=pltpu.CompilerParams(use_tc_tiling_on_sc=False),
    )
    return kern(x2, table)


def _dense_body(z_ref, w1_ref, b1_ref, w2_ref, b2_ref, o_ref):
    z1 = jnp.dot(z_ref[...], w1_ref[...], preferred_element_type=jnp.float32)
    z1 = z1 + b1_ref[...]
    z2 = jnp.dot(z1, w2_ref[...], preferred_element_type=jnp.float32)
    o_ref[...] = z2 + b2_ref[...]


def kernel(x, table, W1, b1, W2, b2):
    x2 = x.reshape(2 * BATCH, HALF)
    z = _pool(x2, table)
    out = pl.pallas_call(
        _dense_body,
        out_shape=jax.ShapeDtypeStruct((BATCH, W2.shape[1]), jnp.float32),
    )(z, W1, b1.reshape(1, -1), W2, b2.reshape(1, -1))
    return out


# speedup vs baseline: 1.1370x; 1.1370x over previous
import functools

import jax
import jax.numpy as jnp
from jax import lax
from jax.experimental import pallas as pl
from jax.experimental.pallas import tpu as pltpu
from jax.experimental.pallas import tpu_sc as plsc

BATCH = 4096
SEQ = 200
EMBED = 64
HALF = SEQ // 2
NC, NS = 2, 16
NW = NC * NS
BPW = BATCH // NW


_UNROLL = 8


def _issue(table_hbm, idx_v, buf, sem, r):
    pltpu.async_copy(table_hbm.at[idx_v.at[2 * r]], buf.at[pl.ds(0, HALF)], sem)
    pltpu.async_copy(
        table_hbm.at[idx_v.at[2 * r + 1]], buf.at[pl.ds(HALF, HALF)], sem
    )


def _drain(table_hbm, idx_v, buf, sem, r):
    pltpu.make_async_copy(
        table_hbm.at[idx_v.at[2 * r]], buf.at[pl.ds(0, HALF)], sem
    ).wait()
    pltpu.make_async_copy(
        table_hbm.at[idx_v.at[2 * r + 1]], buf.at[pl.ds(HALF, HALF)], sem
    ).wait()


def _reduce_row(buf, zacc, r):
    def red(i, acc):
        accs = list(acc)
        for u in range(_UNROLL):
            row = i * _UNROLL + u
            for c in range(4):
                accs[c] = accs[c] + buf[row, pl.ds(16 * c, 16)]
        return tuple(accs)

    zero = jnp.zeros((16,), jnp.float32)
    acc = lax.fori_loop(0, SEQ // _UNROLL, red, (zero,) * 4)
    scale = jnp.float32(1.0 / SEQ)
    for c in range(4):
        zacc[r, pl.ds(16 * c, 16)] = acc[c] * scale


def _pool_body(x_hbm, table_hbm, z_hbm, idx_v, buf0, buf1, zacc, sem0, sem1):
    wid = lax.axis_index("s") * NC + lax.axis_index("c")
    base = wid * BPW
    pltpu.sync_copy(x_hbm.at[pl.ds(2 * base, 2 * BPW)], idx_v)

    _issue(table_hbm, idx_v, buf0, sem0, 0)

    def pair(k, carry):
        r = 2 * k
        _issue(table_hbm, idx_v, buf1, sem1, r + 1)
        _drain(table_hbm, idx_v, buf0, sem0, r)
        _reduce_row(buf0, zacc, r)

        @pl.when(r + 2 < BPW)
        def _():
            _issue(table_hbm, idx_v, buf0, sem0, r + 2)

        _drain(table_hbm, idx_v, buf1, sem1, r + 1)
        _reduce_row(buf1, zacc, r + 1)
        return carry

    lax.fori_loop(0, BPW // 2, pair, 0)
    pltpu.sync_copy(zacc, z_hbm.at[pl.ds(base, BPW)])


@functools.partial(jax.jit, static_argnames=())
def _pool(x2, table):
    mesh = plsc.VectorSubcoreMesh(core_axis_name="c", subcore_axis_name="s")
    kern = pl.kernel(
        _pool_body,
        out_type=jax.ShapeDtypeStruct((BATCH, EMBED), jnp.float32),
        mesh=mesh,
        scratch_types=[
            pltpu.VMEM((2 * BPW, HALF), jnp.int32),
            pltpu.VMEM((SEQ, EMBED), jnp.float32),
            pltpu.VMEM((SEQ, EMBED), jnp.float32),
            pltpu.VMEM((BPW, EMBED), jnp.float32),
            pltpu.SemaphoreType.DMA,
            pltpu.SemaphoreType.DMA,
        ],
        compiler_params=pltpu.CompilerParams(use_tc_tiling_on_sc=False),
    )
    return kern(x2, table)


def _dense_body(z_ref, w1_ref, b1_ref, w2_ref, b2_ref, o_ref):
    z1 = jnp.dot(z_ref[...], w1_ref[...], preferred_element_type=jnp.float32)
    z1 = z1 + b1_ref[...]
    z2 = jnp.dot(z1, w2_ref[...], preferred_element_type=jnp.float32)
    o_ref[...] = z2 + b2_ref[...]


def kernel(x, table, W1, b1, W2, b2):
    x2 = x.reshape(2 * BATCH, HALF)
    z = _pool(x2, table)
    out = pl.pallas_call(
        _dense_body,
        out_shape=jax.ShapeDtypeStruct((BATCH, W2.shape[1]), jnp.float32),
    )(z, W1, b1.reshape(1, -1), W2, b2.reshape(1, -1))
    return out
